# baseline (speedup 1.0000x reference)
import math

import jax
import jax.numpy as jnp
from jax.experimental import pallas as pl
from jax.experimental.pallas import tpu as pltpu

H = 8
D = 128
DD = D // H
B, T, N = 8, 12, 325
SN = int(2 * math.log(N, 2))
MEDIAM = (N - SN) // 2
NS = SN // 2
NTOP = N - SN - MEDIAM
NBTM = MEDIAM
GP = 160
S = SN + 2 * NS

_HI = jax.lax.Precision.HIGHEST
_DF = jax.lax.Precision.DEFAULT


def _dot(a, b, prec=_DF):
    return jnp.dot(a, b, precision=prec)


def _dot_t(a, b, prec=_DF):
    return jax.lax.dot_general(a, b, (((1,), (1,)), ((), ())), precision=prec)


def _softmax_rows(x):
    m = jnp.max(x, axis=1, keepdims=True)
    e = jnp.exp(x - m)
    return e / jnp.sum(e, axis=1, keepdims=True)


def _fwd(q_ref, k_ref, v_ref, m_in_ref,
         wo_ref, bo_ref, wsk_ref, bsk_ref, wsv_ref, bsv_ref,
         lnw_ref, lnb_ref, wf1_ref, bf1_ref, wf2_ref, bf2_ref,
         gt_ref, gb_ref, out_ref, a_ref, m_ref):
    f32 = jnp.float32
    i32 = jnp.int32
    Q = q_ref[0, 0]
    K = k_ref[0, 0]
    V = v_ref[0, 0]
    M3 = m_in_ref[0, 0]
    neg = jnp.float32(-jnp.inf)

    Mc = M3[:, :, None]
    Mr = M3[:, None, :]
    sub_n = jax.lax.broadcasted_iota(i32, (1, N, N), 1)
    lane_m = jax.lax.broadcasted_iota(i32, (1, N, N), 2)
    cmp = (Mc > Mr) | ((Mc == Mr) & (lane_m < sub_n))
    rank = jnp.sum(cmp.astype(f32), axis=2)

    mem_t = (rank >= MEDIAM) & (rank <= N - SN - 1)
    mem_b = rank <= MEDIAM - 1
    giota = jax.lax.broadcasted_iota(i32, (1, 1, GP), 2).astype(f32)
    gt3 = gt_ref[0, 0][:, None, :]
    gb3 = gb_ref[0, 0][:, None, :]
    sel_t = ((rank - MEDIAM)[:, :, None] == giota) & mem_t[:, :, None]
    sel_b = (rank[:, :, None] == giota) & mem_b[:, :, None]
    gsel_t = jnp.sum(jnp.where(sel_t, gt3, 0.0), axis=2)
    gsel_b = jnp.sum(jnp.where(sel_b, gb3, 0.0), axis=2)

    clipM = jnp.maximum(M3, 0.0)
    sc = jnp.where(mem_t, clipM + gsel_t,
                   jnp.where(mem_b, clipM + gsel_b, neg))
    rid = jnp.where(mem_t, 1.0, jnp.where(mem_b, 2.0, 0.0))
    gt_mat = (sc[:, None, :] > sc[:, :, None]) & (rid[:, None, :] == rid[:, :, None])
    cnt = jnp.sum(gt_mat.astype(f32), axis=2)

    slot = jnp.where(rank >= N - SN, (N - 1) - rank, -1.0)
    slot = jnp.where(mem_t & (cnt < NS), SN + cnt, slot)
    slot = jnp.where(mem_b & (cnt < NS), SN + NS + cnt, slot)

    siota = jax.lax.broadcasted_iota(i32, (S, N), 0).astype(f32)
    qr_l, kr_l, val_l = [], [], []
    for h in range(H):
        hs = slice(h * DD, (h + 1) * DD)
        Qh, Kh, Vh = Q[:, hs], K[:, hs], V[:, hs]
        P = (siota == slot[h:h + 1, :]).astype(f32)
        Qr = _dot(P, Qh, _HI)
        Kr = _dot(P, Kh, _HI)
        attn = _softmax_rows(_dot_t(Qr, Kh) * 0.25)
        qr_l.append(Qr)
        kr_l.append(Kr)
        val_l.append(_dot(attn, Vh))

    proj = jnp.concatenate(qr_l, axis=1)
    kred = jnp.concatenate(kr_l, axis=1)
    valm = jnp.concatenate(val_l, axis=1)
    m_ref[0, 0] = valm

    a_ref[0, 0] = _softmax_rows(_dot_t(Q, kred) * 0.25)

    sK = _dot(proj, wsk_ref[...]) + bsk_ref[...]
    sV = _dot(valm, wsv_ref[...]) + bsv_ref[...]
    v2_l = []
    for h in range(H):
        hs = slice(h * DD, (h + 1) * DD)
        sqk = _softmax_rows(_dot_t(Q[:, hs], sK[:, hs]) * 0.25)
        v2_l.append(_dot(sqk, sV[:, hs]))
    v2 = jnp.concatenate(v2_l, axis=1)
    v2 = _dot(v2, wo_ref[...]) + bo_ref[...]
    mu = jnp.mean(v2, axis=1, keepdims=True)
    var = jnp.mean((v2 - mu) ** 2, axis=1, keepdims=True)
    v2 = (v2 - mu) * jax.lax.rsqrt(var + 1e-5) * lnw_ref[...] + lnb_ref[...]
    hdd = jnp.maximum(_dot(v2, wf1_ref[...]) + bf1_ref[...], 0.0)
    ffo = _dot(hdd, wf2_ref[...]) + bf2_ref[...]
    res = v2 + ffo
    mu2 = jnp.mean(res, axis=1, keepdims=True)
    var2 = jnp.mean((res - mu2) ** 2, axis=1, keepdims=True)
    out_ref[0, 0] = (res - mu2) * jax.lax.rsqrt(var2 + 1e-5)


def kernel(x, Wq, bq, Wk, bk, Wv, bv, Wo, bo, Wsk, bsk, Wsv, bsv, Wproj, bproj,
           ln_w, ln_b, Wff1, bff1, Wff2, bff2, statica):
    f32 = jnp.float32

    Q = x @ Wq + bq
    K = x @ Wk + bk
    V = x @ Wv + bv
    Qh = jnp.concatenate(jnp.split(Q, H, axis=-1), axis=0)
    Kh = jnp.concatenate(jnp.split(K, H, axis=-1), axis=0)
    stat = jnp.broadcast_to(statica, (N, statica.shape[1]))
    K_sample = Kh[:, :, stat, :]
    QK_sample = jnp.einsum('btnd,btnsd->btns', Qh, K_sample)
    M = (QK_sample @ Wproj + bproj)[..., 0]
    M4 = M.reshape(H, B, T, N).transpose(1, 2, 0, 3)

    kk = jax.random.key(1234)
    k1, k2 = jax.random.split(kk)
    g_t = jax.random.gumbel(k1, (B * H * T, NTOP), dtype=f32)
    g_b = jax.random.gumbel(k2, (B * H * T, NBTM), dtype=f32)
    g_t = g_t.reshape(H, B, T, NTOP).transpose(1, 2, 0, 3)
    g_b = g_b.reshape(H, B, T, NBTM).transpose(1, 2, 0, 3)
    g_t = jnp.pad(g_t, ((0, 0), (0, 0), (0, 0), (0, GP - NTOP)))
    g_b = jnp.pad(g_b, ((0, 0), (0, 0), (0, 0), (0, GP - NBTM)))

    b2 = lambda v: v.reshape(1, D)
    wspec = pl.BlockSpec((D, D), lambda b, t: (0, 0))
    bspec = pl.BlockSpec((1, D), lambda b, t: (0, 0))
    xspec = pl.BlockSpec((1, 1, N, D), lambda b, t: (b, t, 0, 0))
    gspec = pl.BlockSpec((1, 1, H, GP), lambda b, t: (b, t, 0, 0))

    out, a_pro, m_pro = pl.pallas_call(
        _fwd,
        grid=(B, T),
        in_specs=[
            xspec, xspec, xspec,
            pl.BlockSpec((1, 1, H, N), lambda b, t: (b, t, 0, 0)),
            wspec, bspec, wspec, bspec, wspec, bspec,
            bspec, bspec, wspec, bspec, wspec, bspec,
            gspec, gspec,
        ],
        out_specs=[
            pl.BlockSpec((1, 1, N, D), lambda b, t: (b, t, 0, 0)),
            pl.BlockSpec((1, 1, N, S), lambda b, t: (b, t, 0, 0)),
            pl.BlockSpec((1, 1, S, D), lambda b, t: (b, t, 0, 0)),
        ],
        out_shape=[
            jax.ShapeDtypeStruct((B, T, N, D), f32),
            jax.ShapeDtypeStruct((B, T, N, S), f32),
            jax.ShapeDtypeStruct((B, T, S, D), f32),
        ],
        compiler_params=pltpu.CompilerParams(
            dimension_semantics=("arbitrary", "arbitrary"),
        ),
    )(Q, K, V, M4, Wo, b2(bo), Wsk, b2(bsk), Wsv, b2(bsv),
      b2(ln_w), b2(ln_b), Wff1, b2(bff1), Wff2, b2(bff2), g_t, g_b)

    return out, a_pro, m_pro


# speedup vs baseline: 7.9235x; 7.9235x over previous
import math

import jax
import jax.numpy as jnp
from jax.experimental import pallas as pl
from jax.experimental.pallas import tpu as pltpu

H = 8
D = 128
DD = D // H
B, T, N = 8, 12, 325
SN = int(2 * math.log(N, 2))
MEDIAM = (N - SN) // 2
NS = SN // 2
NTOP = N - SN - MEDIAM
NBTM = MEDIAM
GP = 160
S = SN + 2 * NS

_HI = jax.lax.Precision.HIGHEST
_DF = jax.lax.Precision.DEFAULT


def _dot(a, b, prec=_DF):
    return jnp.dot(a, b, precision=prec)


def _dot_t(a, b, prec=_DF):
    return jax.lax.dot_general(a, b, (((1,), (1,)), ((), ())), precision=prec)


def _softmax_rows(x):
    m = jnp.max(x, axis=1, keepdims=True)
    e = jnp.exp(x - m)
    return e / jnp.sum(e, axis=1, keepdims=True)


def _fwd(q_ref, k_ref, v_ref, m_in_ref,
         wo_ref, bo_ref, wsk_ref, bsk_ref, wsv_ref, bsv_ref,
         lnw_ref, lnb_ref, wf1_ref, bf1_ref, wf2_ref, bf2_ref,
         gt_ref, gb_ref, out_ref, a_ref, m_ref):
    f32 = jnp.float32
    i32 = jnp.int32
    Q = q_ref[0, 0]
    K = k_ref[0, 0]
    V = v_ref[0, 0]
    M3 = m_in_ref[0, 0]
    neg = jnp.float32(-jnp.inf)

    sub_m = jax.lax.broadcasted_iota(i32, (N, N), 0)
    lane_n = jax.lax.broadcasted_iota(i32, (N, N), 1)
    giota = jax.lax.broadcasted_iota(i32, (GP, N), 0).astype(f32)
    siota = jax.lax.broadcasted_iota(i32, (S, N), 0).astype(f32)

    qr_l, kr_l, val_l = [], [], []
    for h in range(H):
        hs = slice(h * DD, (h + 1) * DD)
        Qh, Kh, Vh = Q[:, hs], K[:, hs], V[:, hs]
        Mr = M3[h:h + 1, :]
        Mc = Mr.T
        cmp = (Mc < Mr) | ((Mc == Mr) & (sub_m < lane_n))
        rank = jnp.sum(cmp.astype(f32), axis=0, keepdims=True)

        mem_t = (rank >= MEDIAM) & (rank <= N - SN - 1)
        mem_b = rank <= MEDIAM - 1
        g_t = gt_ref[0, 0][:, h:h + 1]
        g_b = gb_ref[0, 0][:, h:h + 1]
        sel_t = ((rank - MEDIAM) == giota) & mem_t
        sel_b = (rank == giota) & mem_b
        gsel_t = jnp.sum(jnp.where(sel_t, g_t, 0.0), axis=0, keepdims=True)
        gsel_b = jnp.sum(jnp.where(sel_b, g_b, 0.0), axis=0, keepdims=True)

        clipM = jnp.maximum(Mr, 0.0)
        sc = jnp.where(mem_t, clipM + gsel_t,
                       jnp.where(mem_b, clipM + gsel_b, neg))
        rid = jnp.where(mem_t, 1.0, jnp.where(mem_b, 2.0, 0.0))
        gt_mat = (sc.T > sc) & (rid.T == rid)
        cnt = jnp.sum(gt_mat.astype(f32), axis=0, keepdims=True)

        slot = jnp.where(rank >= N - SN, (N - 1) - rank, -1.0)
        slot = jnp.where(mem_t & (cnt < NS), SN + cnt, slot)
        slot = jnp.where(mem_b & (cnt < NS), SN + NS + cnt, slot)

        P = (siota == slot).astype(f32)
        Qr = _dot(P, Qh, _HI)
        Kr = _dot(P, Kh, _HI)
        attn = _softmax_rows(_dot_t(Qr, Kh) * 0.25)
        qr_l.append(Qr)
        kr_l.append(Kr)
        val_l.append(_dot(attn, Vh))

    proj = jnp.concatenate(qr_l, axis=1)
    kred = jnp.concatenate(kr_l, axis=1)
    valm = jnp.concatenate(val_l, axis=1)
    m_ref[0, 0] = valm

    a_ref[0, 0] = _softmax_rows(_dot_t(Q, kred) * 0.25)

    sK = _dot(proj, wsk_ref[...]) + bsk_ref[...]
    sV = _dot(valm, wsv_ref[...]) + bsv_ref[...]
    v2_l = []
    for h in range(H):
        hs = slice(h * DD, (h + 1) * DD)
        sqk = _softmax_rows(_dot_t(Q[:, hs], sK[:, hs]) * 0.25)
        v2_l.append(_dot(sqk, sV[:, hs]))
    v2 = jnp.concatenate(v2_l, axis=1)
    v2 = _dot(v2, wo_ref[...]) + bo_ref[...]
    mu = jnp.mean(v2, axis=1, keepdims=True)
    var = jnp.mean((v2 - mu) ** 2, axis=1, keepdims=True)
    v2 = (v2 - mu) * jax.lax.rsqrt(var + 1e-5) * lnw_ref[...] + lnb_ref[...]
    hdd = jnp.maximum(_dot(v2, wf1_ref[...]) + bf1_ref[...], 0.0)
    ffo = _dot(hdd, wf2_ref[...]) + bf2_ref[...]
    res = v2 + ffo
    mu2 = jnp.mean(res, axis=1, keepdims=True)
    var2 = jnp.mean((res - mu2) ** 2, axis=1, keepdims=True)
    out_ref[0, 0] = (res - mu2) * jax.lax.rsqrt(var2 + 1e-5)


def kernel(x, Wq, bq, Wk, bk, Wv, bv, Wo, bo, Wsk, bsk, Wsv, bsv, Wproj, bproj,
           ln_w, ln_b, Wff1, bff1, Wff2, bff2, statica):
    f32 = jnp.float32

    Q = x @ Wq + bq
    K = x @ Wk + bk
    V = x @ Wv + bv
    Qh = jnp.concatenate(jnp.split(Q, H, axis=-1), axis=0)
    Kh = jnp.concatenate(jnp.split(K, H, axis=-1), axis=0)
    stat = jnp.broadcast_to(statica, (N, statica.shape[1]))
    K_sample = Kh[:, :, stat, :]
    QK_sample = jnp.einsum('btnd,btnsd->btns', Qh, K_sample)
    M = (QK_sample @ Wproj + bproj)[..., 0]
    M4 = M.reshape(H, B, T, N).transpose(1, 2, 0, 3)

    kk = jax.random.key(1234)
    k1, k2 = jax.random.split(kk)
    g_t = jax.random.gumbel(k1, (B * H * T, NTOP), dtype=f32)
    g_b = jax.random.gumbel(k2, (B * H * T, NBTM), dtype=f32)
    g_t = g_t.reshape(H, B, T, NTOP).transpose(1, 2, 3, 0)
    g_b = g_b.reshape(H, B, T, NBTM).transpose(1, 2, 3, 0)
    g_t = jnp.pad(g_t, ((0, 0), (0, 0), (0, GP - NTOP), (0, 0)))
    g_b = jnp.pad(g_b, ((0, 0), (0, 0), (0, GP - NBTM), (0, 0)))

    b2 = lambda v: v.reshape(1, D)
    wspec = pl.BlockSpec((D, D), lambda b, t: (0, 0))
    bspec = pl.BlockSpec((1, D), lambda b, t: (0, 0))
    xspec = pl.BlockSpec((1, 1, N, D), lambda b, t: (b, t, 0, 0))
    gspec = pl.BlockSpec((1, 1, GP, H), lambda b, t: (b, t, 0, 0))

    out, a_pro, m_pro = pl.pallas_call(
        _fwd,
        grid=(B, T),
        in_specs=[
            xspec, xspec, xspec,
            pl.BlockSpec((1, 1, H, N), lambda b, t: (b, t, 0, 0)),
            wspec, bspec, wspec, bspec, wspec, bspec,
            bspec, bspec, wspec, bspec, wspec, bspec,
            gspec, gspec,
        ],
        out_specs=[
            pl.BlockSpec((1, 1, N, D), lambda b, t: (b, t, 0, 0)),
            pl.BlockSpec((1, 1, N, S), lambda b, t: (b, t, 0, 0)),
            pl.BlockSpec((1, 1, S, D), lambda b, t: (b, t, 0, 0)),
        ],
        out_shape=[
            jax.ShapeDtypeStruct((B, T, N, D), f32),
            jax.ShapeDtypeStruct((B, T, N, S), f32),
            jax.ShapeDtypeStruct((B, T, S, D), f32),
        ],
        compiler_params=pltpu.CompilerParams(
            dimension_semantics=("arbitrary", "arbitrary"),
        ),
    )(Q, K, V, M4, Wo, b2(bo), Wsk, b2(bsk), Wsv, b2(bsv),
      b2(ln_w), b2(ln_b), Wff1, b2(bff1), Wff2, b2(bff2), g_t, g_b)

    return out, a_pro, m_pro
